# baseline (speedup 1.0000x reference)
import functools

import jax
import jax.numpy as jnp
from jax import lax
from jax.experimental import pallas as pl
from jax.experimental.pallas import tpu as pltpu
from jax.experimental.pallas import tpu_sc as plsc

B, N, S, D1, D2 = 8, 4096, 1024, 128, 256
C0, C1 = 256, 256
BN_TOT = B * N

NB1 = 512
NB4 = 2048

NW = 32
CHUNK = 32
NCHUNKS = BN_TOT // CHUNK


def _knn_body(x1_ref, x2t_ref, p2_ref, w0_ref, gidx_ref, w_ref, t_ref,
              sq2_ref):
    b = pl.program_id(0)
    i = pl.program_id(1)

    @pl.when(i == 0)
    def _():
        x2t0 = x2t_ref[0]
        sq2_ref[...] = jnp.sum(x2t0 * x2t0, axis=1, keepdims=True)

    for k in range(3):
        @pl.when(i == k)
        def _():
            p2 = p2_ref[0]
            w0k = w0_ref[:, D1 + k * D2:D1 + (k + 1) * D2]
            t_ref[0, 0] = lax.dot_general(
                p2, w0k, (((0,), (1,)), ((), ())),
                preferred_element_type=jnp.float32)

    x1b = x1_ref[0]
    x2t = x2t_ref[0]
    sq1 = jnp.sum(x1b * x1b, axis=0)[None, :]
    cross = lax.dot_general(x2t, x1b, (((1,), (0,)), ((), ())),
                            preferred_element_type=jnp.float32)
    d = -2.0 * cross + sq1 + sq2_ref[...]
    iota0 = lax.broadcasted_iota(jnp.int32, (S, NB1), 0)
    recips = []
    idxs = []
    for k in range(3):
        m = jnp.min(d, axis=0)
        a = jnp.argmin(d, axis=0)
        idxs.append(a)
        recips.append(1.0 / (m + 1e-8))
        if k < 2:
            d = jnp.where(iota0 == a[None, :], jnp.inf, d)
    norm = recips[0] + recips[1] + recips[2]
    for k in range(3):
        gidx_ref[0, k, :] = idxs[k] + (b * 3 + k) * S
        w_ref[0, k, :] = recips[k] / norm


def _knn(xyz1, xyz2t, points2, w0):
    return pl.pallas_call(
        _knn_body,
        grid=(B, N // NB1),
        in_specs=[
            pl.BlockSpec((1, 3, NB1), lambda b, i: (b, 0, i)),
            pl.BlockSpec((1, S, 3), lambda b, i: (b, 0, 0)),
            pl.BlockSpec((1, D2, S), lambda b, i: (b, 0, 0)),
            pl.BlockSpec((C0, D1 + 3 * D2), lambda b, i: (0, 0)),
        ],
        out_specs=[
            pl.BlockSpec((1, 3, NB1), lambda b, i: (b, 0, i)),
            pl.BlockSpec((1, 3, NB1), lambda b, i: (b, 0, i)),
            pl.BlockSpec((1, 1, S, C0),
                         lambda b, i: (b, jnp.minimum(i, 2), 0, 0)),
        ],
        out_shape=[
            jax.ShapeDtypeStruct((B, 3, N), jnp.int32),
            jax.ShapeDtypeStruct((B, 3, N), jnp.float32),
            jax.ShapeDtypeStruct((B, 3, S, C0), jnp.float32),
        ],
        scratch_shapes=[pltpu.VMEM((S, 1), jnp.float32)],
    )(xyz1, xyz2t, points2, w0)


PPW = BN_TOT // NW
NCH = PPW // CHUNK
NPB = N // PPW


def _sc_interp(tflat, gidx, w):
    mesh = plsc.VectorSubcoreMesh(core_axis_name="c", subcore_axis_name="s")

    @functools.partial(
        pl.kernel,
        mesh=mesh,
        out_type=jax.ShapeDtypeStruct((BN_TOT, C0), jnp.float32),
        scratch_types=[
            pltpu.VMEM((3, PPW), jnp.int32),
            pltpu.VMEM((3, PPW), jnp.float32),
            pltpu.VMEM((2, 3, CHUNK, C0), jnp.float32),
            pltpu.VMEM((2, CHUNK, C0), jnp.float32),
            pltpu.SemaphoreType.DMA,
            pltpu.SemaphoreType.DMA,
            pltpu.SemaphoreType.DMA,
            pltpu.SemaphoreType.DMA,
        ],
    )
    def body(t_hbm, g_hbm, w_hbm, out_hbm, idx_v, w_v, rows_v, outb_v,
             semg0, semg1, semo0, semo1):
        cid = lax.axis_index("c")
        sid = lax.axis_index("s")
        wid = sid * 2 + cid
        b = wid // NPB
        nlo = (wid % NPB) * PPW
        base_pt = wid * PPW
        pltpu.sync_copy(g_hbm.at[b, :, pl.ds(nlo, PPW)], idx_v)
        pltpu.sync_copy(w_hbm.at[b, :, pl.ds(nlo, PPW)], w_v)
        semg = (semg0, semg1)
        semo = (semo0, semo1)

        def fire(c, buf):
            for k in range(3):
                iref = idx_v.at[k, pl.ds(c * CHUNK, CHUNK)]
                pltpu.async_copy(t_hbm.at[iref], rows_v.at[buf, k], semg[buf])

        def wait_gather(buf):
            for k in range(3):
                pltpu.make_async_copy(
                    t_hbm.at[pl.ds(0, CHUNK)], rows_v.at[buf, k],
                    semg[buf]).wait()

        def wait_out(buf):
            pltpu.make_async_copy(
                t_hbm.at[pl.ds(0, CHUNK)], outb_v.at[buf], semo[buf]).wait()

        def compute(c, buf):
            wks = [[w_v[k, pl.ds(c * CHUNK + g * 16, 16)]
                    for g in range(CHUNK // 16)] for k in range(3)]
            for j in range(CHUNK):
                jj = jnp.full((16,), j % 16, jnp.int32)
                wvs = [wks[k][j // 16].at[jj].get(mode="promise_in_bounds")
                       for k in range(3)]
                for t in range(C0 // 16):
                    sl = pl.ds(t * 16, 16)
                    outb_v[buf, j, sl] = (wvs[0] * rows_v[buf, 0, j, sl]
                                          + wvs[1] * rows_v[buf, 1, j, sl]
                                          + wvs[2] * rows_v[buf, 2, j, sl])
            pltpu.async_copy(
                outb_v.at[buf],
                out_hbm.at[pl.ds(base_pt + c * CHUNK, CHUNK)], semo[buf])

        fire(0, 0)

        def step(i, carry):
            fire(2 * i + 1, 1)
            wait_gather(0)

            @pl.when(i > 0)
            def _():
                wait_out(0)

            compute(2 * i, 0)

            @pl.when(i < NCH // 2 - 1)
            def _():
                fire(2 * i + 2, 0)

            wait_gather(1)

            @pl.when(i > 0)
            def _():
                wait_out(1)

            compute(2 * i + 1, 1)
            return carry

        lax.fori_loop(0, NCH // 2, step, 0)
        wait_out(0)
        wait_out(1)

    return body(tflat, gidx, w)


def _mlp1_body(it_ref, p1_ref, w_ref, b_ref, h_ref, s_ref, q_ref):
    b = pl.program_id(0)
    i = pl.program_id(1)
    p1b = p1_ref[0]
    w0p = w_ref[...]
    h = lax.dot_general(p1b, w0p, (((0,), (1,)), ((), ())),
                        preferred_element_type=jnp.float32)
    h = h + it_ref[0] + b_ref[...]

    @pl.when(jnp.logical_and(b == 0, i == 0))
    def _():
        s_ref[...] = jnp.zeros_like(s_ref)
        q_ref[...] = jnp.zeros_like(q_ref)

    h_ref[0] = h
    s_ref[...] += jnp.sum(h, axis=0, keepdims=True)
    q_ref[...] += jnp.sum(h * h, axis=0, keepdims=True)


def _mlp1(interp3, points1, w0p, b0):
    return pl.pallas_call(
        _mlp1_body,
        grid=(B, N // NB4),
        in_specs=[
            pl.BlockSpec((1, NB4, C0), lambda b, i: (b, i, 0)),
            pl.BlockSpec((1, D1, NB4), lambda b, i: (b, 0, i)),
            pl.BlockSpec((C0, D1), lambda b, i: (0, 0)),
            pl.BlockSpec((1, C0), lambda b, i: (0, 0)),
        ],
        out_specs=[
            pl.BlockSpec((1, NB4, C0), lambda b, i: (b, i, 0)),
            pl.BlockSpec((1, C0), lambda b, i: (0, 0)),
            pl.BlockSpec((1, C0), lambda b, i: (0, 0)),
        ],
        out_shape=[
            jax.ShapeDtypeStruct((B, N, C0), jnp.float32),
            jax.ShapeDtypeStruct((1, C0), jnp.float32),
            jax.ShapeDtypeStruct((1, C0), jnp.float32),
        ],
    )(interp3, points1, w0p, b0)


def _mlp2_body(h_ref, s_ref, q_ref, g_ref, beta_ref, w_ref, b_ref,
               h1_ref, s1_ref, q1_ref):
    b = pl.program_id(0)
    i = pl.program_id(1)
    cnt = jnp.float32(BN_TOT)
    mean = s_ref[...] / cnt
    var = q_ref[...] / cnt - mean * mean
    inv = lax.rsqrt(var + 1e-5) * g_ref[...]
    t = jnp.maximum((h_ref[0] - mean) * inv + beta_ref[...], 0.0)
    h1 = lax.dot_general(t, w_ref[...], (((1,), (1,)), ((), ())),
                         preferred_element_type=jnp.float32)
    h1 = h1 + b_ref[...]

    @pl.when(jnp.logical_and(b == 0, i == 0))
    def _():
        s1_ref[...] = jnp.zeros_like(s1_ref)
        q1_ref[...] = jnp.zeros_like(q1_ref)

    h1_ref[0] = h1
    s1_ref[...] += jnp.sum(h1, axis=0, keepdims=True)
    q1_ref[...] += jnp.sum(h1 * h1, axis=0, keepdims=True)


def _mlp2(h0, s0, q0, g0, beta0, w1, b1):
    return pl.pallas_call(
        _mlp2_body,
        grid=(B, N // NB4),
        in_specs=[
            pl.BlockSpec((1, NB4, C0), lambda b, i: (b, i, 0)),
            pl.BlockSpec((1, C0), lambda b, i: (0, 0)),
            pl.BlockSpec((1, C0), lambda b, i: (0, 0)),
            pl.BlockSpec((1, C0), lambda b, i: (0, 0)),
            pl.BlockSpec((1, C0), lambda b, i: (0, 0)),
            pl.BlockSpec((C1, C0), lambda b, i: (0, 0)),
            pl.BlockSpec((1, C1), lambda b, i: (0, 0)),
        ],
        out_specs=[
            pl.BlockSpec((1, NB4, C1), lambda b, i: (b, i, 0)),
            pl.BlockSpec((1, C1), lambda b, i: (0, 0)),
            pl.BlockSpec((1, C1), lambda b, i: (0, 0)),
        ],
        out_shape=[
            jax.ShapeDtypeStruct((B, N, C1), jnp.float32),
            jax.ShapeDtypeStruct((1, C1), jnp.float32),
            jax.ShapeDtypeStruct((1, C1), jnp.float32),
        ],
    )(h0, s0, q0, g0, beta0, w1, b1)


def _out_body(h_ref, s_ref, q_ref, g_ref, beta_ref, o_ref):
    cnt = jnp.float32(BN_TOT)
    mean = s_ref[...] / cnt
    var = q_ref[...] / cnt - mean * mean
    inv = lax.rsqrt(var + 1e-5) * g_ref[...]
    t = jnp.maximum((h_ref[0] - mean) * inv + beta_ref[...], 0.0)
    o_ref[0] = t.T


def _outk(h1, s1, q1, g1, beta1):
    return pl.pallas_call(
        _out_body,
        grid=(B, N // NB4),
        in_specs=[
            pl.BlockSpec((1, NB4, C1), lambda b, i: (b, i, 0)),
            pl.BlockSpec((1, C1), lambda b, i: (0, 0)),
            pl.BlockSpec((1, C1), lambda b, i: (0, 0)),
            pl.BlockSpec((1, C1), lambda b, i: (0, 0)),
            pl.BlockSpec((1, C1), lambda b, i: (0, 0)),
        ],
        out_specs=pl.BlockSpec((1, C1, NB4), lambda b, i: (b, 0, i)),
        out_shape=jax.ShapeDtypeStruct((B, C1, N), jnp.float32),
    )(h1, s1, q1, g1, beta1)


def kernel(xyz1, xyz2, points1, points2, W0, b0, g0, beta0, W1, b1, g1, beta1):
    gidx, w, tbl = _knn(xyz1, jnp.transpose(xyz2, (0, 2, 1)), points2, W0)

    interp = _sc_interp(tbl.reshape(B * 3 * S, C0), gidx, w)
    interp3 = interp.reshape(B, N, C0)

    h0, s0, q0 = _mlp1(interp3, points1, W0, b0[None, :])
    h1, s1, q1 = _mlp2(h0, s0, q0, g0[None, :], beta0[None, :], W1, b1[None, :])
    return _outk(h1, s1, q1, g1[None, :], beta1[None, :])


# speedup vs baseline: 1.0595x; 1.0595x over previous
import functools

import jax
import jax.numpy as jnp
from jax import lax
from jax.experimental import pallas as pl
from jax.experimental.pallas import tpu as pltpu
from jax.experimental.pallas import tpu_sc as plsc

B, N, S, D1, D2 = 8, 4096, 1024, 128, 256
C0, C1 = 256, 256
BN_TOT = B * N

NB1 = 512
NB4 = 2048

NW = 32
CHUNK = 16
NCHUNKS = BN_TOT // CHUNK


def _knn_body(x1_ref, x2t_ref, p2_ref, w0_ref, gidx_ref, w_ref, t_ref,
              sq2_ref):
    b = pl.program_id(0)
    i = pl.program_id(1)

    @pl.when(i == 0)
    def _():
        x2t0 = x2t_ref[0]
        sq2_ref[...] = jnp.sum(x2t0 * x2t0, axis=1, keepdims=True)

    for k in range(3):
        @pl.when(i == k)
        def _():
            p2 = p2_ref[0]
            w0k = w0_ref[:, D1 + k * D2:D1 + (k + 1) * D2]
            t_ref[0, 0] = lax.dot_general(
                p2, w0k, (((0,), (1,)), ((), ())),
                preferred_element_type=jnp.float32)

    x1b = x1_ref[0]
    x2t = x2t_ref[0]
    sq1 = jnp.sum(x1b * x1b, axis=0)[None, :]
    cross = lax.dot_general(x2t, x1b, (((1,), (0,)), ((), ())),
                            preferred_element_type=jnp.float32)
    d = -2.0 * cross + sq1 + sq2_ref[...]
    iota0 = lax.broadcasted_iota(jnp.int32, (S, NB1), 0)
    recips = []
    idxs = []
    for k in range(3):
        m = jnp.min(d, axis=0)
        a = jnp.argmin(d, axis=0)
        idxs.append(a)
        recips.append(1.0 / (m + 1e-8))
        if k < 2:
            d = jnp.where(iota0 == a[None, :], jnp.inf, d)
    norm = recips[0] + recips[1] + recips[2]
    for k in range(3):
        gidx_ref[0, k, :] = idxs[k] + (b * 3 + k) * S
        w_ref[0, k, :] = recips[k] / norm


def _knn(xyz1, xyz2t, points2, w0):
    return pl.pallas_call(
        _knn_body,
        grid=(B, N // NB1),
        in_specs=[
            pl.BlockSpec((1, 3, NB1), lambda b, i: (b, 0, i)),
            pl.BlockSpec((1, S, 3), lambda b, i: (b, 0, 0)),
            pl.BlockSpec((1, D2, S), lambda b, i: (b, 0, 0)),
            pl.BlockSpec((C0, D1 + 3 * D2), lambda b, i: (0, 0)),
        ],
        out_specs=[
            pl.BlockSpec((1, 3, NB1), lambda b, i: (b, 0, i)),
            pl.BlockSpec((1, 3, NB1), lambda b, i: (b, 0, i)),
            pl.BlockSpec((1, 1, S, C0),
                         lambda b, i: (b, jnp.minimum(i, 2), 0, 0)),
        ],
        out_shape=[
            jax.ShapeDtypeStruct((B, 3, N), jnp.int32),
            jax.ShapeDtypeStruct((B, 3, N), jnp.float32),
            jax.ShapeDtypeStruct((B, 3, S, C0), jnp.float32),
        ],
        scratch_shapes=[pltpu.VMEM((S, 1), jnp.float32)],
    )(xyz1, xyz2t, points2, w0)


PPW = BN_TOT // NW
NCH = PPW // CHUNK
NPB = N // PPW


def _sc_interp(tflat, gidx, w):
    mesh = plsc.VectorSubcoreMesh(core_axis_name="c", subcore_axis_name="s")

    @functools.partial(
        pl.kernel,
        mesh=mesh,
        out_type=jax.ShapeDtypeStruct((BN_TOT, C0), jnp.float32),
        scratch_types=[
            pltpu.VMEM((3, PPW), jnp.int32),
            pltpu.VMEM((3, PPW), jnp.float32),
            pltpu.VMEM((2, 3, CHUNK, C0), jnp.float32),
            pltpu.VMEM((2, CHUNK, C0), jnp.float32),
            pltpu.SemaphoreType.DMA,
            pltpu.SemaphoreType.DMA,
            pltpu.SemaphoreType.DMA,
            pltpu.SemaphoreType.DMA,
        ],
    )
    def body(t_hbm, g_hbm, w_hbm, out_hbm, idx_v, w_v, rows_v, outb_v,
             semg0, semg1, semo0, semo1):
        cid = lax.axis_index("c")
        sid = lax.axis_index("s")
        wid = sid * 2 + cid
        b = wid // NPB
        nlo = (wid % NPB) * PPW
        base_pt = wid * PPW
        pltpu.sync_copy(g_hbm.at[b, :, pl.ds(nlo, PPW)], idx_v)
        pltpu.sync_copy(w_hbm.at[b, :, pl.ds(nlo, PPW)], w_v)
        semg = (semg0, semg1)
        semo = (semo0, semo1)

        def fire(c, buf):
            for k in range(3):
                ivec = idx_v[k, pl.ds(c * CHUNK, CHUNK)]
                pltpu.async_copy(t_hbm.at[ivec], rows_v.at[buf, k], semg[buf])

        def wait_gather(buf):
            for k in range(3):
                pltpu.make_async_copy(
                    t_hbm.at[pl.ds(0, CHUNK)], rows_v.at[buf, k],
                    semg[buf]).wait()

        def wait_out(buf):
            pltpu.make_async_copy(
                t_hbm.at[pl.ds(0, CHUNK)], outb_v.at[buf], semo[buf]).wait()

        def compute(c, buf):
            wks = [[w_v[k, pl.ds(c * CHUNK + g * 16, 16)]
                    for g in range(CHUNK // 16)] for k in range(3)]
            for j in range(CHUNK):
                jj = jnp.full((16,), j % 16, jnp.int32)
                wvs = [wks[k][j // 16].at[jj].get(mode="promise_in_bounds")
                       for k in range(3)]
                for t in range(C0 // 16):
                    sl = pl.ds(t * 16, 16)
                    outb_v[buf, j, sl] = (wvs[0] * rows_v[buf, 0, j, sl]
                                          + wvs[1] * rows_v[buf, 1, j, sl]
                                          + wvs[2] * rows_v[buf, 2, j, sl])
            pltpu.async_copy(
                outb_v.at[buf],
                out_hbm.at[pl.ds(base_pt + c * CHUNK, CHUNK)], semo[buf])

        fire(0, 0)

        def step(i, carry):
            fire(2 * i + 1, 1)
            wait_gather(0)

            @pl.when(i > 0)
            def _():
                wait_out(0)

            compute(2 * i, 0)

            @pl.when(i < NCH // 2 - 1)
            def _():
                fire(2 * i + 2, 0)

            wait_gather(1)

            @pl.when(i > 0)
            def _():
                wait_out(1)

            compute(2 * i + 1, 1)
            return carry

        lax.fori_loop(0, NCH // 2, step, 0)
        wait_out(0)
        wait_out(1)

    return body(tflat, gidx, w)


def _mlp1_body(it_ref, p1_ref, w_ref, b_ref, h_ref, s_ref, q_ref):
    b = pl.program_id(0)
    i = pl.program_id(1)
    p1b = p1_ref[0]
    w0p = w_ref[...]
    h = lax.dot_general(p1b, w0p, (((0,), (1,)), ((), ())),
                        preferred_element_type=jnp.float32)
    h = h + it_ref[0] + b_ref[...]

    @pl.when(jnp.logical_and(b == 0, i == 0))
    def _():
        s_ref[...] = jnp.zeros_like(s_ref)
        q_ref[...] = jnp.zeros_like(q_ref)

    h_ref[0] = h
    s_ref[...] += jnp.sum(h, axis=0, keepdims=True)
    q_ref[...] += jnp.sum(h * h, axis=0, keepdims=True)


def _mlp1(interp3, points1, w0p, b0):
    return pl.pallas_call(
        _mlp1_body,
        grid=(B, N // NB4),
        in_specs=[
            pl.BlockSpec((1, NB4, C0), lambda b, i: (b, i, 0)),
            pl.BlockSpec((1, D1, NB4), lambda b, i: (b, 0, i)),
            pl.BlockSpec((C0, D1), lambda b, i: (0, 0)),
            pl.BlockSpec((1, C0), lambda b, i: (0, 0)),
        ],
        out_specs=[
            pl.BlockSpec((1, NB4, C0), lambda b, i: (b, i, 0)),
            pl.BlockSpec((1, C0), lambda b, i: (0, 0)),
            pl.BlockSpec((1, C0), lambda b, i: (0, 0)),
        ],
        out_shape=[
            jax.ShapeDtypeStruct((B, N, C0), jnp.float32),
            jax.ShapeDtypeStruct((1, C0), jnp.float32),
            jax.ShapeDtypeStruct((1, C0), jnp.float32),
        ],
    )(interp3, points1, w0p, b0)


def _mlp2_body(h_ref, s_ref, q_ref, g_ref, beta_ref, w_ref, b_ref,
               h1_ref, s1_ref, q1_ref):
    b = pl.program_id(0)
    i = pl.program_id(1)
    cnt = jnp.float32(BN_TOT)
    mean = s_ref[...] / cnt
    var = q_ref[...] / cnt - mean * mean
    inv = lax.rsqrt(var + 1e-5) * g_ref[...]
    t = jnp.maximum((h_ref[0] - mean) * inv + beta_ref[...], 0.0)
    h1 = lax.dot_general(t, w_ref[...], (((1,), (1,)), ((), ())),
                         preferred_element_type=jnp.float32)
    h1 = h1 + b_ref[...]

    @pl.when(jnp.logical_and(b == 0, i == 0))
    def _():
        s1_ref[...] = jnp.zeros_like(s1_ref)
        q1_ref[...] = jnp.zeros_like(q1_ref)

    h1_ref[0] = h1
    s1_ref[...] += jnp.sum(h1, axis=0, keepdims=True)
    q1_ref[...] += jnp.sum(h1 * h1, axis=0, keepdims=True)


def _mlp2(h0, s0, q0, g0, beta0, w1, b1):
    return pl.pallas_call(
        _mlp2_body,
        grid=(B, N // NB4),
        in_specs=[
            pl.BlockSpec((1, NB4, C0), lambda b, i: (b, i, 0)),
            pl.BlockSpec((1, C0), lambda b, i: (0, 0)),
            pl.BlockSpec((1, C0), lambda b, i: (0, 0)),
            pl.BlockSpec((1, C0), lambda b, i: (0, 0)),
            pl.BlockSpec((1, C0), lambda b, i: (0, 0)),
            pl.BlockSpec((C1, C0), lambda b, i: (0, 0)),
            pl.BlockSpec((1, C1), lambda b, i: (0, 0)),
        ],
        out_specs=[
            pl.BlockSpec((1, NB4, C1), lambda b, i: (b, i, 0)),
            pl.BlockSpec((1, C1), lambda b, i: (0, 0)),
            pl.BlockSpec((1, C1), lambda b, i: (0, 0)),
        ],
        out_shape=[
            jax.ShapeDtypeStruct((B, N, C1), jnp.float32),
            jax.ShapeDtypeStruct((1, C1), jnp.float32),
            jax.ShapeDtypeStruct((1, C1), jnp.float32),
        ],
    )(h0, s0, q0, g0, beta0, w1, b1)


def _out_body(h_ref, s_ref, q_ref, g_ref, beta_ref, o_ref):
    cnt = jnp.float32(BN_TOT)
    mean = s_ref[...] / cnt
    var = q_ref[...] / cnt - mean * mean
    inv = lax.rsqrt(var + 1e-5) * g_ref[...]
    t = jnp.maximum((h_ref[0] - mean) * inv + beta_ref[...], 0.0)
    o_ref[0] = t.T


def _outk(h1, s1, q1, g1, beta1):
    return pl.pallas_call(
        _out_body,
        grid=(B, N // NB4),
        in_specs=[
            pl.BlockSpec((1, NB4, C1), lambda b, i: (b, i, 0)),
            pl.BlockSpec((1, C1), lambda b, i: (0, 0)),
            pl.BlockSpec((1, C1), lambda b, i: (0, 0)),
            pl.BlockSpec((1, C1), lambda b, i: (0, 0)),
            pl.BlockSpec((1, C1), lambda b, i: (0, 0)),
        ],
        out_specs=pl.BlockSpec((1, C1, NB4), lambda b, i: (b, 0, i)),
        out_shape=jax.ShapeDtypeStruct((B, C1, N), jnp.float32),
    )(h1, s1, q1, g1, beta1)


def kernel(xyz1, xyz2, points1, points2, W0, b0, g0, beta0, W1, b1, g1, beta1):
    gidx, w, tbl = _knn(xyz1, jnp.transpose(xyz2, (0, 2, 1)), points2, W0)

    interp = _sc_interp(tbl.reshape(B * 3 * S, C0), gidx, w)
    interp3 = interp.reshape(B, N, C0)

    h0, s0, q0 = _mlp1(interp3, points1, W0, b0[None, :])
    h1, s1, q1 = _mlp2(h0, s0, q0, g0[None, :], beta0[None, :], W1, b1[None, :])
    return _outk(h1, s1, q1, g1[None, :], beta1[None, :])


# speedup vs baseline: 1.2114x; 1.1434x over previous
import functools

import jax
import jax.numpy as jnp
from jax import lax
from jax.experimental import pallas as pl
from jax.experimental.pallas import tpu as pltpu
from jax.experimental.pallas import tpu_sc as plsc

B, N, S, D1, D2 = 8, 4096, 1024, 128, 256
C0, C1 = 256, 256
BN_TOT = B * N

NB1 = 512
NB4 = 2048

NW = 32
CHUNK = 16
NCHUNKS = BN_TOT // CHUNK


def _knn_body(x1_ref, x2t_ref, p2_ref, w0_ref, gidx_ref, w_ref, t_ref,
              sq2_ref):
    b = pl.program_id(0)
    i = pl.program_id(1)

    @pl.when(i == 0)
    def _():
        x2t0 = x2t_ref[0]
        sq2_ref[...] = jnp.sum(x2t0 * x2t0, axis=1, keepdims=True)

    for k in range(3):
        @pl.when(i == k)
        def _():
            p2 = p2_ref[0]
            w0k = w0_ref[:, D1 + k * D2:D1 + (k + 1) * D2]
            t_ref[0, 0] = lax.dot_general(
                p2, w0k, (((0,), (1,)), ((), ())),
                preferred_element_type=jnp.float32)

    x1b = x1_ref[0]
    x2t = x2t_ref[0]
    sq1 = jnp.sum(x1b * x1b, axis=0)[None, :]
    cross = lax.dot_general(x2t, x1b, (((1,), (0,)), ((), ())),
                            preferred_element_type=jnp.float32)
    d = -2.0 * cross + sq1 + sq2_ref[...]
    iota0 = lax.broadcasted_iota(jnp.int32, (S, NB1), 0)
    recips = []
    idxs = []
    for k in range(3):
        m = jnp.min(d, axis=0)
        a = jnp.argmin(d, axis=0)
        idxs.append(a)
        recips.append(1.0 / (m + 1e-8))
        if k < 2:
            d = jnp.where(iota0 == a[None, :], jnp.inf, d)
    norm = recips[0] + recips[1] + recips[2]
    for k in range(3):
        gidx_ref[0, k, :] = idxs[k] + (b * 3 + k) * S
        w_ref[0, k, :] = recips[k] / norm


def _knn(xyz1, xyz2t, points2, w0):
    return pl.pallas_call(
        _knn_body,
        grid=(B, N // NB1),
        in_specs=[
            pl.BlockSpec((1, 3, NB1), lambda b, i: (b, 0, i)),
            pl.BlockSpec((1, S, 3), lambda b, i: (b, 0, 0)),
            pl.BlockSpec((1, D2, S), lambda b, i: (b, 0, 0)),
            pl.BlockSpec((C0, D1 + 3 * D2), lambda b, i: (0, 0)),
        ],
        out_specs=[
            pl.BlockSpec((1, 3, NB1), lambda b, i: (b, 0, i)),
            pl.BlockSpec((1, 3, NB1), lambda b, i: (b, 0, i)),
            pl.BlockSpec((1, 1, S, C0),
                         lambda b, i: (b, jnp.minimum(i, 2), 0, 0)),
        ],
        out_shape=[
            jax.ShapeDtypeStruct((B, 3, N), jnp.int32),
            jax.ShapeDtypeStruct((B, 3, N), jnp.float32),
            jax.ShapeDtypeStruct((B, 3, S, C0), jnp.float32),
        ],
        scratch_shapes=[pltpu.VMEM((S, 1), jnp.float32)],
    )(xyz1, xyz2t, points2, w0)


PPW = BN_TOT // NW
NCH = PPW // CHUNK
NPB = N // PPW


def _sc_interp(tflat, gidx, w):
    mesh = plsc.VectorSubcoreMesh(core_axis_name="c", subcore_axis_name="s")

    @functools.partial(
        pl.kernel,
        mesh=mesh,
        out_type=jax.ShapeDtypeStruct((BN_TOT, C0), jnp.float32),
        scratch_types=[
            pltpu.VMEM((3, PPW), jnp.int32),
            pltpu.VMEM((3, PPW), jnp.float32),
            pltpu.VMEM((2, 3, CHUNK, C0), jnp.float32),
            pltpu.VMEM((2, CHUNK, C0), jnp.float32),
            pltpu.SemaphoreType.DMA,
            pltpu.SemaphoreType.DMA,
            pltpu.SemaphoreType.DMA,
            pltpu.SemaphoreType.DMA,
        ],
    )
    def body(t_hbm, g_hbm, w_hbm, out_hbm, idx_v, w_v, rows_v, outb_v,
             semg0, semg1, semo0, semo1):
        cid = lax.axis_index("c")
        sid = lax.axis_index("s")
        wid = sid * 2 + cid
        b = wid // NPB
        nlo = (wid % NPB) * PPW
        base_pt = wid * PPW
        pltpu.sync_copy(g_hbm.at[b, :, pl.ds(nlo, PPW)], idx_v)
        pltpu.sync_copy(w_hbm.at[b, :, pl.ds(nlo, PPW)], w_v)
        semg = (semg0, semg1)
        semo = (semo0, semo1)

        def fire(c, buf):
            for k in range(3):
                ivec = idx_v[k, pl.ds(c * CHUNK, CHUNK)]
                pltpu.async_copy(t_hbm.at[ivec], rows_v.at[buf, k], semg[buf])

        def wait_gather(buf):
            for k in range(3):
                pltpu.make_async_copy(
                    t_hbm.at[pl.ds(0, CHUNK)], rows_v.at[buf, k],
                    semg[buf]).wait()

        def wait_out(buf):
            pltpu.make_async_copy(
                t_hbm.at[pl.ds(0, CHUNK)], outb_v.at[buf], semo[buf]).wait()

        def compute(c, buf):
            for j in range(CHUNK):
                jj = jnp.full((16,), j, jnp.int32)
                wvs = []
                for k in range(3):
                    wk = w_v[k, pl.ds(c * CHUNK, CHUNK)]
                    wvs.append(wk.at[jj].get(mode="promise_in_bounds"))
                for t in range(C0 // 16):
                    sl = pl.ds(t * 16, 16)
                    outb_v[buf, j, sl] = (wvs[0] * rows_v[buf, 0, j, sl]
                                          + wvs[1] * rows_v[buf, 1, j, sl]
                                          + wvs[2] * rows_v[buf, 2, j, sl])
            pltpu.async_copy(
                outb_v.at[buf],
                out_hbm.at[pl.ds(base_pt + c * CHUNK, CHUNK)], semo[buf])

        fire(0, 0)

        def step(i, carry):
            fire(2 * i + 1, 1)
            wait_gather(0)

            @pl.when(i > 0)
            def _():
                wait_out(0)

            compute(2 * i, 0)

            @pl.when(i < NCH // 2 - 1)
            def _():
                fire(2 * i + 2, 0)

            wait_gather(1)

            @pl.when(i > 0)
            def _():
                wait_out(1)

            compute(2 * i + 1, 1)
            return carry

        lax.fori_loop(0, NCH // 2, step, 0)
        wait_out(0)
        wait_out(1)

    return body(tflat, gidx, w)


def _mlp1_body(it_ref, p1_ref, w_ref, b_ref, h_ref, s_ref, q_ref):
    b = pl.program_id(0)
    i = pl.program_id(1)
    p1b = p1_ref[0]
    w0p = w_ref[...]
    h = lax.dot_general(p1b, w0p, (((0,), (1,)), ((), ())),
                        preferred_element_type=jnp.float32)
    h = h + it_ref[0] + b_ref[...]

    @pl.when(jnp.logical_and(b == 0, i == 0))
    def _():
        s_ref[...] = jnp.zeros_like(s_ref)
        q_ref[...] = jnp.zeros_like(q_ref)

    h_ref[0] = h
    s_ref[...] += jnp.sum(h, axis=0, keepdims=True)
    q_ref[...] += jnp.sum(h * h, axis=0, keepdims=True)


def _mlp1(interp3, points1, w0p, b0):
    return pl.pallas_call(
        _mlp1_body,
        grid=(B, N // NB4),
        in_specs=[
            pl.BlockSpec((1, NB4, C0), lambda b, i: (b, i, 0)),
            pl.BlockSpec((1, D1, NB4), lambda b, i: (b, 0, i)),
            pl.BlockSpec((C0, D1), lambda b, i: (0, 0)),
            pl.BlockSpec((1, C0), lambda b, i: (0, 0)),
        ],
        out_specs=[
            pl.BlockSpec((1, NB4, C0), lambda b, i: (b, i, 0)),
            pl.BlockSpec((1, C0), lambda b, i: (0, 0)),
            pl.BlockSpec((1, C0), lambda b, i: (0, 0)),
        ],
        out_shape=[
            jax.ShapeDtypeStruct((B, N, C0), jnp.float32),
            jax.ShapeDtypeStruct((1, C0), jnp.float32),
            jax.ShapeDtypeStruct((1, C0), jnp.float32),
        ],
    )(interp3, points1, w0p, b0)


def _mlp2_body(h_ref, s_ref, q_ref, g_ref, beta_ref, w_ref, b_ref,
               h1_ref, s1_ref, q1_ref):
    b = pl.program_id(0)
    i = pl.program_id(1)
    cnt = jnp.float32(BN_TOT)
    mean = s_ref[...] / cnt
    var = q_ref[...] / cnt - mean * mean
    inv = lax.rsqrt(var + 1e-5) * g_ref[...]
    t = jnp.maximum((h_ref[0] - mean) * inv + beta_ref[...], 0.0)
    h1 = lax.dot_general(t, w_ref[...], (((1,), (1,)), ((), ())),
                         preferred_element_type=jnp.float32)
    h1 = h1 + b_ref[...]

    @pl.when(jnp.logical_and(b == 0, i == 0))
    def _():
        s1_ref[...] = jnp.zeros_like(s1_ref)
        q1_ref[...] = jnp.zeros_like(q1_ref)

    h1_ref[0] = h1
    s1_ref[...] += jnp.sum(h1, axis=0, keepdims=True)
    q1_ref[...] += jnp.sum(h1 * h1, axis=0, keepdims=True)


def _mlp2(h0, s0, q0, g0, beta0, w1, b1):
    return pl.pallas_call(
        _mlp2_body,
        grid=(B, N // NB4),
        in_specs=[
            pl.BlockSpec((1, NB4, C0), lambda b, i: (b, i, 0)),
            pl.BlockSpec((1, C0), lambda b, i: (0, 0)),
            pl.BlockSpec((1, C0), lambda b, i: (0, 0)),
            pl.BlockSpec((1, C0), lambda b, i: (0, 0)),
            pl.BlockSpec((1, C0), lambda b, i: (0, 0)),
            pl.BlockSpec((C1, C0), lambda b, i: (0, 0)),
            pl.BlockSpec((1, C1), lambda b, i: (0, 0)),
        ],
        out_specs=[
            pl.BlockSpec((1, NB4, C1), lambda b, i: (b, i, 0)),
            pl.BlockSpec((1, C1), lambda b, i: (0, 0)),
            pl.BlockSpec((1, C1), lambda b, i: (0, 0)),
        ],
        out_shape=[
            jax.ShapeDtypeStruct((B, N, C1), jnp.float32),
            jax.ShapeDtypeStruct((1, C1), jnp.float32),
            jax.ShapeDtypeStruct((1, C1), jnp.float32),
        ],
    )(h0, s0, q0, g0, beta0, w1, b1)


def _out_body(h_ref, s_ref, q_ref, g_ref, beta_ref, o_ref):
    cnt = jnp.float32(BN_TOT)
    mean = s_ref[...] / cnt
    var = q_ref[...] / cnt - mean * mean
    inv = lax.rsqrt(var + 1e-5) * g_ref[...]
    t = jnp.maximum((h_ref[0] - mean) * inv + beta_ref[...], 0.0)
    o_ref[0] = t.T


def _outk(h1, s1, q1, g1, beta1):
    return pl.pallas_call(
        _out_body,
        grid=(B, N // NB4),
        in_specs=[
            pl.BlockSpec((1, NB4, C1), lambda b, i: (b, i, 0)),
            pl.BlockSpec((1, C1), lambda b, i: (0, 0)),
            pl.BlockSpec((1, C1), lambda b, i: (0, 0)),
            pl.BlockSpec((1, C1), lambda b, i: (0, 0)),
            pl.BlockSpec((1, C1), lambda b, i: (0, 0)),
        ],
        out_specs=pl.BlockSpec((1, C1, NB4), lambda b, i: (b, 0, i)),
        out_shape=jax.ShapeDtypeStruct((B, C1, N), jnp.float32),
    )(h1, s1, q1, g1, beta1)


def kernel(xyz1, xyz2, points1, points2, W0, b0, g0, beta0, W1, b1, g1, beta1):
    gidx, w, tbl = _knn(xyz1, jnp.transpose(xyz2, (0, 2, 1)), points2, W0)

    interp = _sc_interp(tbl.reshape(B * 3 * S, C0), gidx, w)
    interp3 = interp.reshape(B, N, C0)

    h0, s0, q0 = _mlp1(interp3, points1, W0, b0[None, :])
    h1, s1, q1 = _mlp2(h0, s0, q0, g0[None, :], beta0[None, :], W1, b1[None, :])
    return _outk(h1, s1, q1, g1[None, :], beta1[None, :])


# speedup vs baseline: 1.4213x; 1.1732x over previous
import functools

import jax
import jax.numpy as jnp
from jax import lax
from jax.experimental import pallas as pl
from jax.experimental.pallas import tpu as pltpu
from jax.experimental.pallas import tpu_sc as plsc

B, N, S, D1, D2 = 8, 4096, 1024, 128, 256
C0, C1 = 256, 256
BN_TOT = B * N
HB = B // 2

NB1 = 512
NB4 = 2048

NW = 32
CHUNK = 16


def _knn_body(x1_ref, x2t_ref, p2_ref, w0_ref, gidx_ref, w_ref, t_ref,
              sq2_ref):
    b = pl.program_id(0)
    i = pl.program_id(1)

    @pl.when(i == 0)
    def _():
        x2t0 = x2t_ref[0]
        sq2_ref[...] = jnp.sum(x2t0 * x2t0, axis=1, keepdims=True)

    for k in range(3):
        @pl.when(i == k)
        def _():
            p2 = p2_ref[0]
            w0k = w0_ref[:, D1 + k * D2:D1 + (k + 1) * D2]
            t_ref[0, 0] = lax.dot_general(
                p2, w0k, (((0,), (1,)), ((), ())),
                preferred_element_type=jnp.float32)

    x1b = x1_ref[0]
    x2t = x2t_ref[0]
    sq1 = jnp.sum(x1b * x1b, axis=0)[None, :]
    cross = lax.dot_general(x2t, x1b, (((1,), (0,)), ((), ())),
                            preferred_element_type=jnp.float32)
    d = -2.0 * cross + sq1 + sq2_ref[...]
    iota0 = lax.broadcasted_iota(jnp.int32, (S, NB1), 0)
    recips = []
    idxs = []
    for k in range(3):
        m = jnp.min(d, axis=0)
        a = jnp.argmin(d, axis=0)
        idxs.append(a)
        recips.append(1.0 / (m + 1e-8))
        if k < 2:
            d = jnp.where(iota0 == a[None, :], jnp.inf, d)
    norm = recips[0] + recips[1] + recips[2]
    for k in range(3):
        gidx_ref[0, k, :] = idxs[k] + (b * 3 + k) * S
        w_ref[0, k, :] = recips[k] / norm


def _knn(xyz1, xyz2t, points2, w0, boff):
    return pl.pallas_call(
        _knn_body,
        grid=(HB, N // NB1),
        in_specs=[
            pl.BlockSpec((1, 3, NB1), lambda b, i: (b + boff, 0, i)),
            pl.BlockSpec((1, S, 3), lambda b, i: (b + boff, 0, 0)),
            pl.BlockSpec((1, D2, S), lambda b, i: (b + boff, 0, 0)),
            pl.BlockSpec((C0, D1 + 3 * D2), lambda b, i: (0, 0)),
        ],
        out_specs=[
            pl.BlockSpec((1, 3, NB1), lambda b, i: (b, 0, i)),
            pl.BlockSpec((1, 3, NB1), lambda b, i: (b, 0, i)),
            pl.BlockSpec((1, 1, S, C0),
                         lambda b, i: (b, jnp.minimum(i, 2), 0, 0)),
        ],
        out_shape=[
            jax.ShapeDtypeStruct((HB, 3, N), jnp.int32),
            jax.ShapeDtypeStruct((HB, 3, N), jnp.float32),
            jax.ShapeDtypeStruct((HB, 3, S, C0), jnp.float32),
        ],
        scratch_shapes=[pltpu.VMEM((S, 1), jnp.float32)],
    )(xyz1, xyz2t, points2, w0)


PPW = HB * N // NW
NCH = PPW // CHUNK
NPB = N // PPW


def _sc_interp(tflat, gidx, w):
    mesh = plsc.VectorSubcoreMesh(core_axis_name="c", subcore_axis_name="s")

    @functools.partial(
        pl.kernel,
        mesh=mesh,
        out_type=jax.ShapeDtypeStruct((HB * N, C0), jnp.float32),
        scratch_types=[
            pltpu.VMEM((3, PPW), jnp.int32),
            pltpu.VMEM((3, PPW), jnp.float32),
            pltpu.VMEM((2, 3, CHUNK, C0), jnp.float32),
            pltpu.VMEM((2, CHUNK, C0), jnp.float32),
            pltpu.SemaphoreType.DMA,
            pltpu.SemaphoreType.DMA,
            pltpu.SemaphoreType.DMA,
            pltpu.SemaphoreType.DMA,
        ],
    )
    def body(t_hbm, g_hbm, w_hbm, out_hbm, idx_v, w_v, rows_v, outb_v,
             semg0, semg1, semo0, semo1):
        cid = lax.axis_index("c")
        sid = lax.axis_index("s")
        wid = sid * 2 + cid
        b = wid // NPB
        nlo = (wid % NPB) * PPW
        base_pt = wid * PPW
        pltpu.sync_copy(g_hbm.at[b, :, pl.ds(nlo, PPW)], idx_v)
        pltpu.sync_copy(w_hbm.at[b, :, pl.ds(nlo, PPW)], w_v)
        semg = (semg0, semg1)
        semo = (semo0, semo1)

        def fire(c, buf):
            for k in range(3):
                ivec = idx_v[k, pl.ds(c * CHUNK, CHUNK)]
                pltpu.async_copy(t_hbm.at[ivec], rows_v.at[buf, k], semg[buf])

        def wait_gather(buf):
            for k in range(3):
                pltpu.make_async_copy(
                    t_hbm.at[pl.ds(0, CHUNK)], rows_v.at[buf, k],
                    semg[buf]).wait()

        def wait_out(buf):
            pltpu.make_async_copy(
                t_hbm.at[pl.ds(0, CHUNK)], outb_v.at[buf], semo[buf]).wait()

        def compute(c, buf):
            for j in range(CHUNK):
                jj = jnp.full((16,), j, jnp.int32)
                wvs = []
                for k in range(3):
                    wk = w_v[k, pl.ds(c * CHUNK, CHUNK)]
                    wvs.append(wk.at[jj].get(mode="promise_in_bounds"))
                for t in range(C0 // 16):
                    sl = pl.ds(t * 16, 16)
                    outb_v[buf, j, sl] = (wvs[0] * rows_v[buf, 0, j, sl]
                                          + wvs[1] * rows_v[buf, 1, j, sl]
                                          + wvs[2] * rows_v[buf, 2, j, sl])
            pltpu.async_copy(
                outb_v.at[buf],
                out_hbm.at[pl.ds(base_pt + c * CHUNK, CHUNK)], semo[buf])

        fire(0, 0)

        def step(i, carry):
            fire(2 * i + 1, 1)
            wait_gather(0)

            @pl.when(i > 0)
            def _():
                wait_out(0)

            compute(2 * i, 0)

            @pl.when(i < NCH // 2 - 1)
            def _():
                fire(2 * i + 2, 0)

            wait_gather(1)

            @pl.when(i > 0)
            def _():
                wait_out(1)

            compute(2 * i + 1, 1)
            return carry

        lax.fori_loop(0, NCH // 2, step, 0)
        wait_out(0)
        wait_out(1)

    return body(tflat, gidx, w)


def _mlp1_body(it_ref, p1_ref, w_ref, b_ref, h_ref, s_ref, q_ref):
    b = pl.program_id(0)
    i = pl.program_id(1)
    p1b = p1_ref[0]
    w0p = w_ref[...]
    h = lax.dot_general(p1b, w0p, (((0,), (1,)), ((), ())),
                        preferred_element_type=jnp.float32)
    h = h + it_ref[0] + b_ref[...]

    @pl.when(jnp.logical_and(b == 0, i == 0))
    def _():
        s_ref[...] = jnp.zeros_like(s_ref)
        q_ref[...] = jnp.zeros_like(q_ref)

    h_ref[0] = h
    s_ref[...] += jnp.sum(h, axis=0, keepdims=True)
    q_ref[...] += jnp.sum(h * h, axis=0, keepdims=True)


def _mlp1a_body(it_ref, p1_ref, w_ref, b_ref, h_ref, s_ref, q_ref):
    _mlp1_body(it_ref, p1_ref, w_ref, b_ref, h_ref, s_ref, q_ref)


def _mlp1b_body(it_ref, p1_ref, w_ref, b_ref, hprev_ref, h_ref, s_ref,
                q_ref):
    del hprev_ref
    _mlp1_body(it_ref, p1_ref, w_ref, b_ref, h_ref, s_ref, q_ref)


def _mlp1(interp3, points1, w0, b0, boff, h0_prev=None):
    in_specs = [
        pl.BlockSpec((1, NB4, C0), lambda b, i: (b, i, 0)),
        pl.BlockSpec((1, D1, NB4), lambda b, i: (b + boff, 0, i)),
        pl.BlockSpec((C0, D1), lambda b, i: (0, 0)),
        pl.BlockSpec((1, C0), lambda b, i: (0, 0)),
    ]
    args = [interp3, points1, w0, b0]
    kwargs = {}
    if h0_prev is None:
        body = _mlp1a_body
    else:
        body = _mlp1b_body
        in_specs.append(pl.BlockSpec((1, 8, 128), lambda b, i: (b, 0, 0)))
        args.append(h0_prev)
        kwargs["input_output_aliases"] = {4: 0}
    return pl.pallas_call(
        body,
        grid=(HB, N // NB4),
        in_specs=in_specs,
        out_specs=[
            pl.BlockSpec((1, NB4, C0), lambda b, i: (b + boff, i, 0)),
            pl.BlockSpec((1, C0), lambda b, i: (0, 0)),
            pl.BlockSpec((1, C0), lambda b, i: (0, 0)),
        ],
        out_shape=[
            jax.ShapeDtypeStruct((B, N, C0), jnp.float32),
            jax.ShapeDtypeStruct((1, C0), jnp.float32),
            jax.ShapeDtypeStruct((1, C0), jnp.float32),
        ],
        **kwargs,
    )(*args)


def _mlp2_body(h_ref, s_ref, q_ref, g_ref, beta_ref, w_ref, b_ref,
               h1_ref, s1_ref, q1_ref):
    b = pl.program_id(0)
    i = pl.program_id(1)
    cnt = jnp.float32(BN_TOT)
    mean = s_ref[...] / cnt
    var = q_ref[...] / cnt - mean * mean
    inv = lax.rsqrt(var + 1e-5) * g_ref[...]
    t = jnp.maximum((h_ref[0] - mean) * inv + beta_ref[...], 0.0)
    h1 = lax.dot_general(t, w_ref[...], (((1,), (1,)), ((), ())),
                         preferred_element_type=jnp.float32)
    h1 = h1 + b_ref[...]

    @pl.when(jnp.logical_and(b == 0, i == 0))
    def _():
        s1_ref[...] = jnp.zeros_like(s1_ref)
        q1_ref[...] = jnp.zeros_like(q1_ref)

    h1_ref[0] = h1
    s1_ref[...] += jnp.sum(h1, axis=0, keepdims=True)
    q1_ref[...] += jnp.sum(h1 * h1, axis=0, keepdims=True)


def _mlp2(h0, s0, q0, g0, beta0, w1, b1):
    return pl.pallas_call(
        _mlp2_body,
        grid=(B, N // NB4),
        in_specs=[
            pl.BlockSpec((1, NB4, C0), lambda b, i: (b, i, 0)),
            pl.BlockSpec((1, C0), lambda b, i: (0, 0)),
            pl.BlockSpec((1, C0), lambda b, i: (0, 0)),
            pl.BlockSpec((1, C0), lambda b, i: (0, 0)),
            pl.BlockSpec((1, C0), lambda b, i: (0, 0)),
            pl.BlockSpec((C1, C0), lambda b, i: (0, 0)),
            pl.BlockSpec((1, C1), lambda b, i: (0, 0)),
        ],
        out_specs=[
            pl.BlockSpec((1, NB4, C1), lambda b, i: (b, i, 0)),
            pl.BlockSpec((1, C1), lambda b, i: (0, 0)),
            pl.BlockSpec((1, C1), lambda b, i: (0, 0)),
        ],
        out_shape=[
            jax.ShapeDtypeStruct((B, N, C1), jnp.float32),
            jax.ShapeDtypeStruct((1, C1), jnp.float32),
            jax.ShapeDtypeStruct((1, C1), jnp.float32),
        ],
    )(h0, s0, q0, g0, beta0, w1, b1)


def _out_body(h_ref, s_ref, q_ref, g_ref, beta_ref, o_ref):
    cnt = jnp.float32(BN_TOT)
    mean = s_ref[...] / cnt
    var = q_ref[...] / cnt - mean * mean
    inv = lax.rsqrt(var + 1e-5) * g_ref[...]
    t = jnp.maximum((h_ref[0] - mean) * inv + beta_ref[...], 0.0)
    o_ref[0] = t.T


def _outk(h1, s1, q1, g1, beta1):
    return pl.pallas_call(
        _out_body,
        grid=(B, N // NB4),
        in_specs=[
            pl.BlockSpec((1, NB4, C1), lambda b, i: (b, i, 0)),
            pl.BlockSpec((1, C1), lambda b, i: (0, 0)),
            pl.BlockSpec((1, C1), lambda b, i: (0, 0)),
            pl.BlockSpec((1, C1), lambda b, i: (0, 0)),
            pl.BlockSpec((1, C1), lambda b, i: (0, 0)),
        ],
        out_specs=pl.BlockSpec((1, C1, NB4), lambda b, i: (b, 0, i)),
        out_shape=jax.ShapeDtypeStruct((B, C1, N), jnp.float32),
    )(h1, s1, q1, g1, beta1)


def kernel(xyz1, xyz2, points1, points2, W0, b0, g0, beta0, W1, b1, g1, beta1):
    x2t = jnp.transpose(xyz2, (0, 2, 1))

    ga, wa, ta = _knn(xyz1, x2t, points2, W0, 0)
    gb, wb, tb = _knn(xyz1, x2t, points2, W0, HB)
    ia = _sc_interp(ta.reshape(HB * 3 * S, C0), ga, wa)
    ib = _sc_interp(tb.reshape(HB * 3 * S, C0), gb, wb)

    h0a, s0a, q0a = _mlp1(ia.reshape(HB, N, C0), points1, W0, b0[None, :], 0)
    h0, s0b, q0b = _mlp1(ib.reshape(HB, N, C0), points1, W0, b0[None, :],
                         HB, h0_prev=h0a)
    s0 = s0a + s0b
    q0 = q0a + q0b

    h1, s1, q1 = _mlp2(h0, s0, q0, g0[None, :], beta0[None, :], W1, b1[None, :])
    return _outk(h1, s1, q1, g1[None, :], beta1[None, :])


# speedup vs baseline: 1.5240x; 1.0723x over previous
import functools

import jax
import jax.numpy as jnp
from jax import lax
from jax.experimental import pallas as pl
from jax.experimental.pallas import tpu as pltpu
from jax.experimental.pallas import tpu_sc as plsc

B, N, S, D1, D2 = 8, 4096, 1024, 128, 256
C0, C1 = 256, 256
BN_TOT = B * N
HB = B // 4

NB1 = 512
NB4 = 2048

NW = 32
CHUNK = 16


def _knn_body(x1_ref, x2t_ref, p2_ref, w0_ref, gidx_ref, w_ref, t_ref,
              sq2_ref):
    b = pl.program_id(0)
    i = pl.program_id(1)

    @pl.when(i == 0)
    def _():
        x2t0 = x2t_ref[0]
        sq2_ref[...] = jnp.sum(x2t0 * x2t0, axis=1, keepdims=True)

    for k in range(3):
        @pl.when(i == k)
        def _():
            p2 = p2_ref[0]
            w0k = w0_ref[:, D1 + k * D2:D1 + (k + 1) * D2]
            t_ref[0, 0] = lax.dot_general(
                p2, w0k, (((0,), (1,)), ((), ())),
                preferred_element_type=jnp.float32)

    x1b = x1_ref[0]
    x2t = x2t_ref[0]
    sq1 = jnp.sum(x1b * x1b, axis=0)[None, :]
    cross = lax.dot_general(x2t, x1b, (((1,), (0,)), ((), ())),
                            preferred_element_type=jnp.float32)
    d = -2.0 * cross + sq1 + sq2_ref[...]
    iota0 = lax.broadcasted_iota(jnp.int32, (S, NB1), 0)
    recips = []
    idxs = []
    for k in range(3):
        m = jnp.min(d, axis=0)
        a = jnp.argmin(d, axis=0)
        idxs.append(a)
        recips.append(1.0 / (m + 1e-8))
        if k < 2:
            d = jnp.where(iota0 == a[None, :], jnp.inf, d)
    norm = recips[0] + recips[1] + recips[2]
    for k in range(3):
        gidx_ref[0, k, :] = idxs[k] + (b * 3 + k) * S
        w_ref[0, k, :] = recips[k] / norm


def _knn(xyz1, xyz2t, points2, w0, boff):
    return pl.pallas_call(
        _knn_body,
        grid=(HB, N // NB1),
        in_specs=[
            pl.BlockSpec((1, 3, NB1), lambda b, i: (b + boff, 0, i)),
            pl.BlockSpec((1, S, 3), lambda b, i: (b + boff, 0, 0)),
            pl.BlockSpec((1, D2, S), lambda b, i: (b + boff, 0, 0)),
            pl.BlockSpec((C0, D1 + 3 * D2), lambda b, i: (0, 0)),
        ],
        out_specs=[
            pl.BlockSpec((1, 3, NB1), lambda b, i: (b, 0, i)),
            pl.BlockSpec((1, 3, NB1), lambda b, i: (b, 0, i)),
            pl.BlockSpec((1, 1, S, C0),
                         lambda b, i: (b, jnp.minimum(i, 2), 0, 0)),
        ],
        out_shape=[
            jax.ShapeDtypeStruct((HB, 3, N), jnp.int32),
            jax.ShapeDtypeStruct((HB, 3, N), jnp.float32),
            jax.ShapeDtypeStruct((HB, 3, S, C0), jnp.float32),
        ],
        scratch_shapes=[pltpu.VMEM((S, 1), jnp.float32)],
    )(xyz1, xyz2t, points2, w0)


PPW = HB * N // NW
NCH = PPW // CHUNK
NPB = N // PPW


def _sc_interp(tflat, gidx, w):
    mesh = plsc.VectorSubcoreMesh(core_axis_name="c", subcore_axis_name="s")

    @functools.partial(
        pl.kernel,
        mesh=mesh,
        out_type=jax.ShapeDtypeStruct((HB * N, C0), jnp.float32),
        scratch_types=[
            pltpu.VMEM((3, PPW), jnp.int32),
            pltpu.VMEM((3, PPW), jnp.float32),
            pltpu.VMEM((2, 3, CHUNK, C0), jnp.float32),
            pltpu.VMEM((2, CHUNK, C0), jnp.float32),
            pltpu.SemaphoreType.DMA,
            pltpu.SemaphoreType.DMA,
            pltpu.SemaphoreType.DMA,
            pltpu.SemaphoreType.DMA,
        ],
    )
    def body(t_hbm, g_hbm, w_hbm, out_hbm, idx_v, w_v, rows_v, outb_v,
             semg0, semg1, semo0, semo1):
        cid = lax.axis_index("c")
        sid = lax.axis_index("s")
        wid = sid * 2 + cid
        b = wid // NPB
        nlo = (wid % NPB) * PPW
        base_pt = wid * PPW
        pltpu.sync_copy(g_hbm.at[b, :, pl.ds(nlo, PPW)], idx_v)
        pltpu.sync_copy(w_hbm.at[b, :, pl.ds(nlo, PPW)], w_v)
        semg = (semg0, semg1)
        semo = (semo0, semo1)

        def fire(c, buf):
            for k in range(3):
                ivec = idx_v[k, pl.ds(c * CHUNK, CHUNK)]
                pltpu.async_copy(t_hbm.at[ivec], rows_v.at[buf, k], semg[buf])

        def wait_gather(buf):
            for k in range(3):
                pltpu.make_async_copy(
                    t_hbm.at[pl.ds(0, CHUNK)], rows_v.at[buf, k],
                    semg[buf]).wait()

        def wait_out(buf):
            pltpu.make_async_copy(
                t_hbm.at[pl.ds(0, CHUNK)], outb_v.at[buf], semo[buf]).wait()

        def compute(c, buf):
            for j in range(CHUNK):
                jj = jnp.full((16,), j, jnp.int32)
                wvs = []
                for k in range(3):
                    wk = w_v[k, pl.ds(c * CHUNK, CHUNK)]
                    wvs.append(wk.at[jj].get(mode="promise_in_bounds"))
                for t in range(C0 // 16):
                    sl = pl.ds(t * 16, 16)
                    outb_v[buf, j, sl] = (wvs[0] * rows_v[buf, 0, j, sl]
                                          + wvs[1] * rows_v[buf, 1, j, sl]
                                          + wvs[2] * rows_v[buf, 2, j, sl])
            pltpu.async_copy(
                outb_v.at[buf],
                out_hbm.at[pl.ds(base_pt + c * CHUNK, CHUNK)], semo[buf])

        fire(0, 0)

        def step(i, carry):
            fire(2 * i + 1, 1)
            wait_gather(0)

            @pl.when(i > 0)
            def _():
                wait_out(0)

            compute(2 * i, 0)

            @pl.when(i < NCH // 2 - 1)
            def _():
                fire(2 * i + 2, 0)

            wait_gather(1)

            @pl.when(i > 0)
            def _():
                wait_out(1)

            compute(2 * i + 1, 1)
            return carry

        lax.fori_loop(0, NCH // 2, step, 0)
        wait_out(0)
        wait_out(1)

    return body(tflat, gidx, w)


def _mlp1_body(it_ref, p1_ref, w_ref, b_ref, h_ref, s_ref, q_ref):
    b = pl.program_id(0)
    i = pl.program_id(1)
    p1b = p1_ref[0]
    w0p = w_ref[...]
    h = lax.dot_general(p1b, w0p, (((0,), (1,)), ((), ())),
                        preferred_element_type=jnp.float32)
    h = h + it_ref[0] + b_ref[...]

    @pl.when(jnp.logical_and(b == 0, i == 0))
    def _():
        s_ref[...] = jnp.zeros_like(s_ref)
        q_ref[...] = jnp.zeros_like(q_ref)

    h_ref[0] = h
    s_ref[...] += jnp.sum(h, axis=0, keepdims=True)
    q_ref[...] += jnp.sum(h * h, axis=0, keepdims=True)


def _mlp1a_body(it_ref, p1_ref, w_ref, b_ref, h_ref, s_ref, q_ref):
    _mlp1_body(it_ref, p1_ref, w_ref, b_ref, h_ref, s_ref, q_ref)


def _mlp1b_body(it_ref, p1_ref, w_ref, b_ref, hprev_ref, h_ref, s_ref,
                q_ref):
    del hprev_ref
    _mlp1_body(it_ref, p1_ref, w_ref, b_ref, h_ref, s_ref, q_ref)


def _mlp1(interp3, points1, w0, b0, boff, h0_prev=None):
    in_specs = [
        pl.BlockSpec((1, NB4, C0), lambda b, i: (b, i, 0)),
        pl.BlockSpec((1, D1, NB4), lambda b, i: (b + boff, 0, i)),
        pl.BlockSpec((C0, D1), lambda b, i: (0, 0)),
        pl.BlockSpec((1, C0), lambda b, i: (0, 0)),
    ]
    args = [interp3, points1, w0, b0]
    kwargs = {}
    if h0_prev is None:
        body = _mlp1a_body
    else:
        body = _mlp1b_body
        in_specs.append(pl.BlockSpec((1, 8, 128), lambda b, i: (b, 0, 0)))
        args.append(h0_prev)
        kwargs["input_output_aliases"] = {4: 0}
    return pl.pallas_call(
        body,
        grid=(HB, N // NB4),
        in_specs=in_specs,
        out_specs=[
            pl.BlockSpec((1, NB4, C0), lambda b, i: (b + boff, i, 0)),
            pl.BlockSpec((1, C0), lambda b, i: (0, 0)),
            pl.BlockSpec((1, C0), lambda b, i: (0, 0)),
        ],
        out_shape=[
            jax.ShapeDtypeStruct((B, N, C0), jnp.float32),
            jax.ShapeDtypeStruct((1, C0), jnp.float32),
            jax.ShapeDtypeStruct((1, C0), jnp.float32),
        ],
        **kwargs,
    )(*args)


def _mlp2_body(h_ref, s_ref, q_ref, g_ref, beta_ref, w_ref, b_ref,
               h1_ref, s1_ref, q1_ref):
    b = pl.program_id(0)
    i = pl.program_id(1)
    cnt = jnp.float32(BN_TOT)
    mean = s_ref[...] / cnt
    var = q_ref[...] / cnt - mean * mean
    inv = lax.rsqrt(var + 1e-5) * g_ref[...]
    t = jnp.maximum((h_ref[0] - mean) * inv + beta_ref[...], 0.0)
    h1 = lax.dot_general(t, w_ref[...], (((1,), (1,)), ((), ())),
                         preferred_element_type=jnp.float32)
    h1 = h1 + b_ref[...]

    @pl.when(jnp.logical_and(b == 0, i == 0))
    def _():
        s1_ref[...] = jnp.zeros_like(s1_ref)
        q1_ref[...] = jnp.zeros_like(q1_ref)

    h1_ref[0] = h1
    s1_ref[...] += jnp.sum(h1, axis=0, keepdims=True)
    q1_ref[...] += jnp.sum(h1 * h1, axis=0, keepdims=True)


def _mlp2(h0, s0, q0, g0, beta0, w1, b1):
    return pl.pallas_call(
        _mlp2_body,
        grid=(B, N // NB4),
        in_specs=[
            pl.BlockSpec((1, NB4, C0), lambda b, i: (b, i, 0)),
            pl.BlockSpec((1, C0), lambda b, i: (0, 0)),
            pl.BlockSpec((1, C0), lambda b, i: (0, 0)),
            pl.BlockSpec((1, C0), lambda b, i: (0, 0)),
            pl.BlockSpec((1, C0), lambda b, i: (0, 0)),
            pl.BlockSpec((C1, C0), lambda b, i: (0, 0)),
            pl.BlockSpec((1, C1), lambda b, i: (0, 0)),
        ],
        out_specs=[
            pl.BlockSpec((1, NB4, C1), lambda b, i: (b, i, 0)),
            pl.BlockSpec((1, C1), lambda b, i: (0, 0)),
            pl.BlockSpec((1, C1), lambda b, i: (0, 0)),
        ],
        out_shape=[
            jax.ShapeDtypeStruct((B, N, C1), jnp.float32),
            jax.ShapeDtypeStruct((1, C1), jnp.float32),
            jax.ShapeDtypeStruct((1, C1), jnp.float32),
        ],
    )(h0, s0, q0, g0, beta0, w1, b1)


def _out_body(h_ref, s_ref, q_ref, g_ref, beta_ref, o_ref):
    cnt = jnp.float32(BN_TOT)
    mean = s_ref[...] / cnt
    var = q_ref[...] / cnt - mean * mean
    inv = lax.rsqrt(var + 1e-5) * g_ref[...]
    t = jnp.maximum((h_ref[0] - mean) * inv + beta_ref[...], 0.0)
    o_ref[0] = t.T


def _outk(h1, s1, q1, g1, beta1):
    return pl.pallas_call(
        _out_body,
        grid=(B, N // NB4),
        in_specs=[
            pl.BlockSpec((1, NB4, C1), lambda b, i: (b, i, 0)),
            pl.BlockSpec((1, C1), lambda b, i: (0, 0)),
            pl.BlockSpec((1, C1), lambda b, i: (0, 0)),
            pl.BlockSpec((1, C1), lambda b, i: (0, 0)),
            pl.BlockSpec((1, C1), lambda b, i: (0, 0)),
        ],
        out_specs=pl.BlockSpec((1, C1, NB4), lambda b, i: (b, 0, i)),
        out_shape=jax.ShapeDtypeStruct((B, C1, N), jnp.float32),
    )(h1, s1, q1, g1, beta1)


def kernel(xyz1, xyz2, points1, points2, W0, b0, g0, beta0, W1, b1, g1, beta1):
    x2t = jnp.transpose(xyz2, (0, 2, 1))

    h0 = None
    s0 = q0 = 0.0
    for qi in range(B // HB):
        g, wq, t = _knn(xyz1, x2t, points2, W0, qi * HB)
        ii = _sc_interp(t.reshape(HB * 3 * S, C0), g, wq)
        h0, s, q = _mlp1(ii.reshape(HB, N, C0), points1, W0, b0[None, :],
                         qi * HB, h0_prev=h0)
        s0 = s0 + s
        q0 = q0 + q

    h1, s1, q1 = _mlp2(h0, s0, q0, g0[None, :], beta0[None, :], W1, b1[None, :])
    return _outk(h1, s1, q1, g1[None, :], beta1[None, :])
